# baseline (device time: 14303 ns/iter reference)
import jax
import jax.numpy as jnp
from jax import lax
from jax.experimental import pallas as pl
from jax.experimental.pallas import tpu as pltpu

HALF = 256
S = 48
FWD = HALF - S
FWD_CHUNKS = [(0, 64), (64, 56), (120, 48), (168, 40)]
NC = len(FWD_CHUNKS)
NX = NC + 2


def kernel(x):
    m, n = x.shape

    def body(x_ref, out_ref, xbuf, xsend, xrecv, ysend, yrecv):
        my_x = lax.axis_index("x")
        my_y = lax.axis_index("y")
        my_z = lax.axis_index("z")
        xpeer = (1 - my_x, my_y, my_z)
        ypeer = (my_x, 1 - my_y, my_z)

        own_lo = my_y * HALF
        other_lo = (1 - my_y) * HALF

        barrier_sem = pltpu.get_barrier_semaphore()
        for peer in (xpeer, ypeer):
            pl.semaphore_signal(
                barrier_sem, inc=1, device_id=peer,
                device_id_type=pl.DeviceIdType.MESH,
            )
        pl.semaphore_wait(barrier_sem, 2)

        xr = []
        for i, (off, sz) in enumerate(FWD_CHUNKS):
            r = pltpu.make_async_remote_copy(
                src_ref=x_ref.at[pl.ds(own_lo + off, sz)],
                dst_ref=xbuf.at[pl.ds(off, sz)],
                send_sem=xsend.at[i],
                recv_sem=xrecv.at[i],
                device_id=xpeer,
                device_id_type=pl.DeviceIdType.MESH,
            )
            r.start()
            xr.append(r)
        r = pltpu.make_async_remote_copy(
            src_ref=x_ref.at[pl.ds(own_lo + FWD, S)],
            dst_ref=xbuf.at[pl.ds(FWD, S)],
            send_sem=xsend.at[NC],
            recv_sem=xrecv.at[NC],
            device_id=xpeer,
            device_id_type=pl.DeviceIdType.MESH,
        )
        r.start()
        xr.append(r)
        r = pltpu.make_async_remote_copy(
            src_ref=x_ref.at[pl.ds(other_lo + FWD, S)],
            dst_ref=xbuf.at[pl.ds(HALF, S)],
            send_sem=xsend.at[NC + 1],
            recv_sem=xrecv.at[NC + 1],
            device_id=xpeer,
            device_id_type=pl.DeviceIdType.MESH,
        )
        r.start()
        xr.append(r)

        yr = []
        for i, (off, sz) in enumerate(FWD_CHUNKS):
            xr[i].wait_recv()
            sl = pl.ds(own_lo + off, sz)
            out_ref[sl, :] = x_ref[sl, :] + xbuf[pl.ds(off, sz), :]
            r = pltpu.make_async_remote_copy(
                src_ref=out_ref.at[sl],
                dst_ref=out_ref.at[sl],
                send_sem=ysend.at[i],
                recv_sem=yrecv.at[i],
                device_id=ypeer,
                device_id_type=pl.DeviceIdType.MESH,
            )
            r.start()
            yr.append(r)

        xr[NC].wait_recv()
        sl = pl.ds(own_lo + FWD, S)
        out_ref[sl, :] = x_ref[sl, :] + xbuf[pl.ds(FWD, S), :]

        xr[NC + 1].wait_recv()
        sl = pl.ds(other_lo + FWD, S)
        out_ref[sl, :] = x_ref[sl, :] + xbuf[pl.ds(HALF, S), :]

        for i in range(NC):
            yr[i].wait_recv()
        for i in range(NX):
            xr[i].wait_send()
        for i in range(NC):
            yr[i].wait_send()

    return pl.pallas_call(
        body,
        out_shape=jax.ShapeDtypeStruct((m, n), x.dtype),
        in_specs=[pl.BlockSpec(memory_space=pltpu.VMEM)],
        out_specs=pl.BlockSpec(memory_space=pltpu.VMEM),
        scratch_shapes=[
            pltpu.VMEM((HALF + S, n), x.dtype),
            pltpu.SemaphoreType.DMA((NX,)),
            pltpu.SemaphoreType.DMA((NX,)),
            pltpu.SemaphoreType.DMA((NC,)),
            pltpu.SemaphoreType.DMA((NC,)),
        ],
        compiler_params=pltpu.CompilerParams(collective_id=0),
    )(x)


# device time: 14107 ns/iter; 1.0139x vs baseline; 1.0139x over previous
import jax
import jax.numpy as jnp
from jax import lax
from jax.experimental import pallas as pl
from jax.experimental.pallas import tpu as pltpu

HALF = 256
S = 48
FWD = HALF - S
FWD_CHUNKS = [(0, 48), (48, 40), (88, 32), (120, 32), (152, 24),
              (176, 16), (192, 16)]
NC = len(FWD_CHUNKS)
NX = NC + 2


def kernel(x):
    m, n = x.shape

    def body(x_ref, out_ref, xbuf, xsend, xrecv, ysend, yrecv):
        my_x = lax.axis_index("x")
        my_y = lax.axis_index("y")
        my_z = lax.axis_index("z")
        xpeer = (1 - my_x, my_y, my_z)
        ypeer = (my_x, 1 - my_y, my_z)

        own_lo = my_y * HALF
        other_lo = (1 - my_y) * HALF

        barrier_sem = pltpu.get_barrier_semaphore()
        for peer in (xpeer, ypeer):
            pl.semaphore_signal(
                barrier_sem, inc=1, device_id=peer,
                device_id_type=pl.DeviceIdType.MESH,
            )
        pl.semaphore_wait(barrier_sem, 2)

        xr = []
        for i, (off, sz) in enumerate(FWD_CHUNKS):
            r = pltpu.make_async_remote_copy(
                src_ref=x_ref.at[pl.ds(own_lo + off, sz)],
                dst_ref=xbuf.at[pl.ds(off, sz)],
                send_sem=xsend.at[i],
                recv_sem=xrecv.at[i],
                device_id=xpeer,
                device_id_type=pl.DeviceIdType.MESH,
            )
            r.start()
            xr.append(r)
        r = pltpu.make_async_remote_copy(
            src_ref=x_ref.at[pl.ds(own_lo + FWD, S)],
            dst_ref=xbuf.at[pl.ds(FWD, S)],
            send_sem=xsend.at[NC],
            recv_sem=xrecv.at[NC],
            device_id=xpeer,
            device_id_type=pl.DeviceIdType.MESH,
        )
        r.start()
        xr.append(r)
        r = pltpu.make_async_remote_copy(
            src_ref=x_ref.at[pl.ds(other_lo + FWD, S)],
            dst_ref=xbuf.at[pl.ds(HALF, S)],
            send_sem=xsend.at[NC + 1],
            recv_sem=xrecv.at[NC + 1],
            device_id=xpeer,
            device_id_type=pl.DeviceIdType.MESH,
        )
        r.start()
        xr.append(r)

        yr = []
        for i, (off, sz) in enumerate(FWD_CHUNKS):
            xr[i].wait_recv()
            sl = pl.ds(own_lo + off, sz)
            out_ref[sl, :] = x_ref[sl, :] + xbuf[pl.ds(off, sz), :]
            r = pltpu.make_async_remote_copy(
                src_ref=out_ref.at[sl],
                dst_ref=out_ref.at[sl],
                send_sem=ysend.at[i],
                recv_sem=yrecv.at[i],
                device_id=ypeer,
                device_id_type=pl.DeviceIdType.MESH,
            )
            r.start()
            yr.append(r)

        xr[NC].wait_recv()
        sl = pl.ds(own_lo + FWD, S)
        out_ref[sl, :] = x_ref[sl, :] + xbuf[pl.ds(FWD, S), :]

        xr[NC + 1].wait_recv()
        sl = pl.ds(other_lo + FWD, S)
        out_ref[sl, :] = x_ref[sl, :] + xbuf[pl.ds(HALF, S), :]

        for i in range(NC):
            yr[i].wait_recv()
        for i in range(NX):
            xr[i].wait_send()
        for i in range(NC):
            yr[i].wait_send()

    return pl.pallas_call(
        body,
        out_shape=jax.ShapeDtypeStruct((m, n), x.dtype),
        in_specs=[pl.BlockSpec(memory_space=pltpu.VMEM)],
        out_specs=pl.BlockSpec(memory_space=pltpu.VMEM),
        scratch_shapes=[
            pltpu.VMEM((HALF + S, n), x.dtype),
            pltpu.SemaphoreType.DMA((NX,)),
            pltpu.SemaphoreType.DMA((NX,)),
            pltpu.SemaphoreType.DMA((NC,)),
            pltpu.SemaphoreType.DMA((NC,)),
        ],
        compiler_params=pltpu.CompilerParams(collective_id=0),
    )(x)


# device time: 14044 ns/iter; 1.0184x vs baseline; 1.0045x over previous
import jax
import jax.numpy as jnp
from jax import lax
from jax.experimental import pallas as pl
from jax.experimental.pallas import tpu as pltpu

HALF = 256
S = 48
FWD = HALF - S
FWD_CHUNKS = [(0, 48), (48, 40), (88, 40), (128, 32), (160, 24),
              (184, 16), (200, 8)]
NC = len(FWD_CHUNKS)
NX = NC + 2


def kernel(x):
    m, n = x.shape

    def body(x_ref, out_ref, xbuf, xsend, xrecv, ysend, yrecv):
        my_x = lax.axis_index("x")
        my_y = lax.axis_index("y")
        my_z = lax.axis_index("z")
        xpeer = (1 - my_x, my_y, my_z)
        ypeer = (my_x, 1 - my_y, my_z)

        own_lo = my_y * HALF
        other_lo = (1 - my_y) * HALF

        barrier_sem = pltpu.get_barrier_semaphore()
        for peer in (xpeer, ypeer):
            pl.semaphore_signal(
                barrier_sem, inc=1, device_id=peer,
                device_id_type=pl.DeviceIdType.MESH,
            )
        pl.semaphore_wait(barrier_sem, 2)

        xr = []
        for i, (off, sz) in enumerate(FWD_CHUNKS):
            r = pltpu.make_async_remote_copy(
                src_ref=x_ref.at[pl.ds(own_lo + off, sz)],
                dst_ref=xbuf.at[pl.ds(off, sz)],
                send_sem=xsend.at[i],
                recv_sem=xrecv.at[i],
                device_id=xpeer,
                device_id_type=pl.DeviceIdType.MESH,
            )
            r.start()
            xr.append(r)
        r = pltpu.make_async_remote_copy(
            src_ref=x_ref.at[pl.ds(own_lo + FWD, S)],
            dst_ref=xbuf.at[pl.ds(FWD, S)],
            send_sem=xsend.at[NC],
            recv_sem=xrecv.at[NC],
            device_id=xpeer,
            device_id_type=pl.DeviceIdType.MESH,
        )
        r.start()
        xr.append(r)
        r = pltpu.make_async_remote_copy(
            src_ref=x_ref.at[pl.ds(other_lo + FWD, S)],
            dst_ref=xbuf.at[pl.ds(HALF, S)],
            send_sem=xsend.at[NC + 1],
            recv_sem=xrecv.at[NC + 1],
            device_id=xpeer,
            device_id_type=pl.DeviceIdType.MESH,
        )
        r.start()
        xr.append(r)

        yr = []
        for i, (off, sz) in enumerate(FWD_CHUNKS):
            xr[i].wait_recv()
            sl = pl.ds(own_lo + off, sz)
            out_ref[sl, :] = x_ref[sl, :] + xbuf[pl.ds(off, sz), :]
            r = pltpu.make_async_remote_copy(
                src_ref=out_ref.at[sl],
                dst_ref=out_ref.at[sl],
                send_sem=ysend.at[i],
                recv_sem=yrecv.at[i],
                device_id=ypeer,
                device_id_type=pl.DeviceIdType.MESH,
            )
            r.start()
            yr.append(r)

        xr[NC].wait_recv()
        sl = pl.ds(own_lo + FWD, S)
        out_ref[sl, :] = x_ref[sl, :] + xbuf[pl.ds(FWD, S), :]

        xr[NC + 1].wait_recv()
        sl = pl.ds(other_lo + FWD, S)
        out_ref[sl, :] = x_ref[sl, :] + xbuf[pl.ds(HALF, S), :]

        for i in range(NC):
            yr[i].wait_recv()
        for i in range(NX):
            xr[i].wait_send()
        for i in range(NC):
            yr[i].wait_send()

    return pl.pallas_call(
        body,
        out_shape=jax.ShapeDtypeStruct((m, n), x.dtype),
        in_specs=[pl.BlockSpec(memory_space=pltpu.VMEM)],
        out_specs=pl.BlockSpec(memory_space=pltpu.VMEM),
        scratch_shapes=[
            pltpu.VMEM((HALF + S, n), x.dtype),
            pltpu.SemaphoreType.DMA((NX,)),
            pltpu.SemaphoreType.DMA((NX,)),
            pltpu.SemaphoreType.DMA((NC,)),
            pltpu.SemaphoreType.DMA((NC,)),
        ],
        compiler_params=pltpu.CompilerParams(collective_id=0),
    )(x)
